# baseline (speedup 1.0000x reference)
import jax
import jax.numpy as jnp
from jax import lax
from jax.experimental import pallas as pl
from jax.experimental.pallas import tpu as pltpu

N_TOKENS = 32768
D_IN = 1024
N_EXPERTS = 8
CH = 1024
NBUF = 4


def _router_body(z_hbm, w_ref, b_ref, out_ref, bufs, sems):
    n_chunks = N_TOKENS // CH
    w = w_ref[...]
    b = b_ref[...]

    def start(c):
        i = c % NBUF
        pltpu.make_async_copy(
            z_hbm.at[pl.ds(c * CH, CH), :], bufs.at[i], sems.at[i]).start()

    for c in range(NBUF):
        start(c)
    for c in range(n_chunks):
        i = c % NBUF
        pltpu.make_async_copy(
            z_hbm.at[pl.ds(c * CH, CH), :], bufs.at[i], sems.at[i]).wait()
        z = bufs[i]
        logits = lax.dot_general(z, w, (((1,), (1,)), ((), ())),
                                 preferred_element_type=jnp.float32)
        logits = logits + b
        m = jnp.max(logits, axis=-1, keepdims=True)
        e = jnp.exp(logits - m)
        s = jnp.sum(e, axis=-1, keepdims=True)
        out_ref[pl.ds(c * CH, CH), :] = e / s
        if c + NBUF < n_chunks:
            start(c + NBUF)


def kernel(z, W, b):
    n_tokens = z.shape[0]
    b2 = b.reshape(1, N_EXPERTS)
    return pl.pallas_call(
        _router_body,
        in_specs=[
            pl.BlockSpec(memory_space=pltpu.MemorySpace.HBM),
            pl.BlockSpec((N_EXPERTS, D_IN), lambda: (0, 0)),
            pl.BlockSpec((1, N_EXPERTS), lambda: (0, 0)),
        ],
        out_specs=pl.BlockSpec((n_tokens, N_EXPERTS), lambda: (0, 0)),
        out_shape=jax.ShapeDtypeStruct((n_tokens, N_EXPERTS), jnp.float32),
        scratch_shapes=[
            pltpu.VMEM((NBUF, CH, D_IN), jnp.float32),
            pltpu.SemaphoreType.DMA((NBUF,)),
        ],
    )(z, W, b2)
